# baseline (speedup 1.0000x reference)
import functools

import jax
import jax.numpy as jnp
from jax import lax
from jax.experimental import pallas as pl
from jax.experimental.pallas import tpu as pltpu
from jax.experimental.pallas import tpu_sc as plsc

_NUM_EXPERTS = 8
_TOP_K = 2
_D_MODEL = 768
_N_TOKENS = 32768
_Z_LOSS_COEFF = 0.001
_AUX_LOSS_COEFF = 0.01

_NCHUNK = 2
_CN = _N_TOKENS // _NCHUNK

_TILE = 4096
_GRID = _CN // _TILE

_NC = 2
_NS = 16
_LANES = 16
_NW = _NC * _NS
_CH = _CN // _NW


def _matmul_body(x_ref, w_ref, lg_ref, z_ref, agg_ref, zacc, aggacc):
    step = pl.program_id(0)

    @pl.when(step == 0)
    def _init():
        zacc[...] = jnp.zeros_like(zacc)
        aggacc[...] = jnp.zeros_like(aggacc)

    logits = lax.dot_general(w_ref[...], x_ref[...], (((1,), (1,)), ((), ())),
                             preferred_element_type=jnp.float32)
    lg_ref[...] = logits

    m1 = jnp.max(logits, axis=0, keepdims=True)
    exps = jnp.exp(logits - m1)
    denom = jnp.sum(exps, axis=0, keepdims=True)
    lse = m1 + jnp.log(denom)
    zacc[...] += jnp.sum(lse * lse)
    aggacc[...] += jnp.sum(exps / denom, axis=1, keepdims=True)

    @pl.when(step == _GRID - 1)
    def _fini():
        z_ref[...] = zacc[...]
        agg_ref[...] = aggacc[...]


def _tc_matmul(x, W, chunk):
    return pl.pallas_call(
        _matmul_body,
        grid=(_GRID,),
        in_specs=[
            pl.BlockSpec((_TILE, _D_MODEL),
                         lambda i, c=chunk: (c * _GRID + i, 0)),
            pl.BlockSpec((_NUM_EXPERTS, _D_MODEL), lambda i: (0, 0)),
        ],
        out_specs=[
            pl.BlockSpec((_NUM_EXPERTS, _TILE), lambda i: (0, i)),
            pl.BlockSpec((1, 1), lambda i: (0, 0)),
            pl.BlockSpec((_NUM_EXPERTS, 1), lambda i: (0, 0)),
        ],
        out_shape=[
            jax.ShapeDtypeStruct((_NUM_EXPERTS, _CN), jnp.float32),
            jax.ShapeDtypeStruct((1, 1), jnp.float32),
            jax.ShapeDtypeStruct((_NUM_EXPERTS, 1), jnp.float32),
        ],
        scratch_shapes=[
            pltpu.VMEM((1, 1), jnp.float32),
            pltpu.VMEM((_NUM_EXPERTS, 1), jnp.float32),
        ],
    )(x, W)


_sc_mesh = plsc.VectorSubcoreMesh(core_axis_name="c", subcore_axis_name="s")


@functools.partial(
    pl.kernel,
    mesh=_sc_mesh,
    out_type=[
        jax.ShapeDtypeStruct((_TOP_K, _CN), jnp.float32),
        jax.ShapeDtypeStruct((_TOP_K, _CN), jnp.int32),
        jax.ShapeDtypeStruct((_NW, _NUM_EXPERTS * _LANES), jnp.float32),
    ],
    scratch_types=[
        pltpu.VMEM((_NUM_EXPERTS, _CH), jnp.float32),
        pltpu.VMEM((_TOP_K, _CH), jnp.float32),
        pltpu.VMEM((_TOP_K, _CH), jnp.int32),
        pltpu.VMEM((_NUM_EXPERTS * _LANES,), jnp.float32),
    ],
)
def _sc_route(lg_hbm, wts_hbm, idx_hbm, cnt_hbm, lg_v, w_v, i_v, cnt_v):
    wid = lax.axis_index("s") * _NC + lax.axis_index("c")
    base = wid * _CH
    pltpu.sync_copy(lg_hbm.at[:, pl.ds(base, _CH)], lg_v)

    zeros = jnp.zeros((_LANES,), jnp.float32)

    def body(i, cnt_acc):
        t = i * _LANES
        v = [lg_v[e, pl.ds(t, _LANES)] for e in range(_NUM_EXPERTS)]
        m1 = v[0]
        for e in range(1, _NUM_EXPERTS):
            m1 = jnp.maximum(m1, v[e])
        big = jnp.full((_LANES,), _NUM_EXPERTS, jnp.int32)
        i1 = big
        for e in range(_NUM_EXPERTS - 1, -1, -1):
            i1 = jnp.where(v[e] == m1, jnp.int32(e), i1)
        neg = jnp.float32(-3.0e38)
        m2 = jnp.where(i1 == 0, neg, v[0])
        for e in range(1, _NUM_EXPERTS):
            m2 = jnp.maximum(m2, jnp.where(i1 == e, neg, v[e]))
        i2 = big
        for e in range(_NUM_EXPERTS - 1, -1, -1):
            i2 = jnp.where(jnp.logical_and(v[e] == m2, i1 != e),
                           jnp.int32(e), i2)
        denom = jnp.exp(v[0] - m1)
        for e in range(1, _NUM_EXPERTS):
            denom = denom + jnp.exp(v[e] - m1)
        rden = 1.0 / denom
        w_v[0, pl.ds(t, _LANES)] = rden
        w_v[1, pl.ds(t, _LANES)] = jnp.exp(m2 - m1) * rden
        i_v[0, pl.ds(t, _LANES)] = i1
        i_v[1, pl.ds(t, _LANES)] = i2
        one = jnp.float32(1.0)
        zero = jnp.float32(0.0)
        new_acc = []
        for e in range(_NUM_EXPERTS):
            hits = (jnp.where(i1 == e, one, zero) +
                    jnp.where(i2 == e, one, zero))
            new_acc.append(cnt_acc[e] + hits)
        return tuple(new_acc)

    cnt_acc = lax.fori_loop(
        0, _CH // _LANES, body,
        tuple(zeros for _ in range(_NUM_EXPERTS)))

    for e in range(_NUM_EXPERTS):
        cnt_v[pl.ds(e * _LANES, _LANES)] = cnt_acc[e]

    pltpu.sync_copy(w_v, wts_hbm.at[:, pl.ds(base, _CH)])
    pltpu.sync_copy(i_v, idx_hbm.at[:, pl.ds(base, _CH)])
    pltpu.sync_copy(cnt_v, cnt_hbm.at[wid])


def _fin_body(cnt_a, cnt_b, agg_a, agg_b, z_a, z_b, z_ref, aux_ref):
    s = jnp.sum(cnt_a[...] + cnt_b[...], axis=0, keepdims=True)
    agg = agg_a[...] + agg_b[...]
    eol = lax.broadcasted_iota(jnp.int32, s.shape, 1) // _LANES
    acc = jnp.float32(0.0)
    for e in range(_NUM_EXPERTS):
        acc += agg[e, 0] * jnp.sum(jnp.where(eol == e, s, 0.0))
    aux_scale = _NUM_EXPERTS * _AUX_LOSS_COEFF / (
        float(_N_TOKENS) * float(_N_TOKENS) * _TOP_K)
    aux_ref[...] = jnp.full((1, 1), acc * aux_scale, jnp.float32)
    z_ref[...] = (z_a[...] + z_b[...]) * (_Z_LOSS_COEFF / _N_TOKENS)


def _tc_fin(cnt_a, cnt_b, agg_a, agg_b, z_a, z_b):
    return pl.pallas_call(
        _fin_body,
        out_shape=[
            jax.ShapeDtypeStruct((1, 1), jnp.float32),
            jax.ShapeDtypeStruct((1, 1), jnp.float32),
        ],
    )(cnt_a, cnt_b, agg_a, agg_b, z_a, z_b)


def kernel(x, W):
    lg_a, z_a, agg_a = _tc_matmul(x, W, 0)
    lg_b, z_b, agg_b = _tc_matmul(x, W, 1)
    wts_a, idx_a, cnt_a = _sc_route(lg_a)
    wts_b, idx_b, cnt_b = _sc_route(lg_b)
    z, aux = _tc_fin(cnt_a, cnt_b, agg_a, agg_b, z_a, z_b)
    wts = jnp.concatenate([wts_a, wts_b], axis=1).T
    idx = jnp.concatenate([idx_a, idx_b], axis=1).T
    return wts, idx, z[0, 0], aux[0, 0]


# speedup vs baseline: 1.8707x; 1.8707x over previous
import jax
import jax.numpy as jnp
from jax import lax
from jax.experimental import pallas as pl
from jax.experimental.pallas import tpu as pltpu

_NUM_EXPERTS = 8
_TOP_K = 2
_D_MODEL = 768
_N_TOKENS = 32768
_Z_LOSS_COEFF = 0.001
_AUX_LOSS_COEFF = 0.01

_TILE = 1024
_GRID = _N_TOKENS // _TILE
_NBUF = 4


def _outer_body(x_hbm, w_ref, wts_hbm, idx_hbm, z_ref, aux_ref,
                zacc, agg, cnt):
    zacc[...] = jnp.zeros_like(zacc)
    agg[...] = jnp.zeros_like(agg)
    cnt[...] = jnp.zeros_like(cnt)
    w = w_ref[...]

    def inner(x_ref, wts_ref, idx_ref):
        logits = lax.dot_general(w, x_ref[...], (((1,), (1,)), ((), ())),
                                 preferred_element_type=jnp.float32)

        m1 = jnp.max(logits, axis=0, keepdims=True)
        exps = jnp.exp(logits - m1)
        denom = jnp.sum(exps, axis=0, keepdims=True)
        rdenom = 1.0 / denom
        scores = exps * rdenom

        eids = lax.broadcasted_iota(jnp.int32, logits.shape, 0)
        big = jnp.int32(_NUM_EXPERTS)
        i1 = jnp.min(jnp.where(logits == m1, eids, big), axis=0,
                     keepdims=True)
        masked = jnp.where(eids == i1, -jnp.inf, logits)
        m2 = jnp.max(masked, axis=0, keepdims=True)
        i2 = jnp.min(jnp.where(masked == m2, eids, big), axis=0,
                     keepdims=True)

        w1 = rdenom
        w2 = jnp.exp(m2 - m1) * rdenom
        wts_ref[...] = jnp.concatenate([w1, w2], axis=0)
        idx_ref[...] = jnp.concatenate([i1, i2], axis=0)

        lse = m1 + jnp.log(denom)
        zacc[...] += jnp.sum(lse * lse)
        agg[...] += jnp.sum(scores, axis=1, keepdims=True)
        onehot = (jnp.where(eids == i1, 1.0, 0.0) +
                  jnp.where(eids == i2, 1.0, 0.0))
        cnt[...] += jnp.sum(onehot, axis=1, keepdims=True)

    pipe = pltpu.emit_pipeline(
        inner,
        grid=(_GRID,),
        in_specs=[
            pl.BlockSpec((_TILE, _D_MODEL), lambda i: (i, 0),
                         pipeline_mode=pl.Buffered(buffer_count=_NBUF)),
        ],
        out_specs=[
            pl.BlockSpec((_TOP_K, _TILE), lambda i: (0, i)),
            pl.BlockSpec((_TOP_K, _TILE), lambda i: (0, i)),
        ],
    )
    pipe(x_hbm, wts_hbm, idx_hbm)

    z_ref[...] = zacc[...] * (_Z_LOSS_COEFF / _N_TOKENS)
    aux_scale = _NUM_EXPERTS * _AUX_LOSS_COEFF / (
        float(_N_TOKENS) * float(_N_TOKENS) * _TOP_K)
    aux_ref[...] = jnp.sum(agg[...] * cnt[...]).reshape(1, 1) * aux_scale


def kernel(x, W):
    wts, idx, z, aux = pl.pallas_call(
        _outer_body,
        in_specs=[
            pl.BlockSpec(memory_space=pl.ANY),
            pl.BlockSpec((_NUM_EXPERTS, _D_MODEL), lambda: (0, 0)),
        ],
        out_specs=[
            pl.BlockSpec(memory_space=pl.ANY),
            pl.BlockSpec(memory_space=pl.ANY),
            pl.BlockSpec((1, 1), lambda: (0, 0)),
            pl.BlockSpec((1, 1), lambda: (0, 0)),
        ],
        out_shape=[
            jax.ShapeDtypeStruct((_TOP_K, _N_TOKENS), jnp.float32),
            jax.ShapeDtypeStruct((_TOP_K, _N_TOKENS), jnp.int32),
            jax.ShapeDtypeStruct((1, 1), jnp.float32),
            jax.ShapeDtypeStruct((1, 1), jnp.float32),
        ],
        scratch_shapes=[
            pltpu.VMEM((1, 1), jnp.float32),
            pltpu.VMEM((_NUM_EXPERTS, 1), jnp.float32),
            pltpu.VMEM((_NUM_EXPERTS, 1), jnp.float32),
        ],
    )(x, W)
    return wts.T, idx.T, z[0, 0], aux[0, 0]
